# baseline (speedup 1.0000x reference)
import functools

import jax
import jax.numpy as jnp
import numpy as np
from jax import lax
from jax.experimental import pallas as pl
from jax.experimental.pallas import tpu as pltpu
from jax.experimental.pallas import tpu_sc as plsc
from jax.experimental.shard_map import shard_map
from jax.sharding import Mesh, PartitionSpec as P

_SU, _SL, _TU, _TL = 1000.0, 0.0, 500.0, 0.0
_BB = 2
_DPAD = 128


def _pick_rows_per_worker(n_rows):
    for b_per_w in (128, 64, 32, 16, 8):
        if n_rows % b_per_w == 0 and n_rows // b_per_w <= 32:
            return b_per_w
    return None


def _sc_gather(table, idx, n_rows):
    info = plsc.get_sparse_core_info()
    b_per_w = _pick_rows_per_worker(n_rows)
    n_active = n_rows // b_per_w
    d = table.shape[1]
    mesh = plsc.VectorSubcoreMesh(core_axis_name="c", subcore_axis_name="s")

    @functools.partial(
        pl.kernel, mesh=mesh,
        out_type=jax.ShapeDtypeStruct((n_rows, d), jnp.float32),
        scratch_types=[
            pltpu.VMEM((b_per_w,), jnp.int32),
            pltpu.VMEM((b_per_w, d), jnp.float32),
            pltpu.SemaphoreType.DMA,
        ],
    )
    def k(table_hbm, idx_hbm, out_hbm, idx_v, rows_v, sem):
        wid = lax.axis_index("s") * info.num_cores + lax.axis_index("c")

        @pl.when(wid < n_active)
        def _():
            base = wid * b_per_w
            pltpu.sync_copy(idx_hbm.at[pl.ds(base, b_per_w)], idx_v)
            pltpu.async_copy(table_hbm.at[idx_v], rows_v, sem).wait()
            pltpu.sync_copy(rows_v, out_hbm.at[pl.ds(base, b_per_w)])

    return k(table, idx)


def _expand_kernel(lmax_ref,
                   ds_ref, tlen_ref, su_ref, sl_ref, tu_ref, tl_ref,
                   out_ref):
    bb, m_sz, l, e = out_ref.shape
    rows = bb * m_sz

    lmax = lmax_ref[0]
    m_pp = jax.lax.broadcasted_iota(jnp.int32, (rows, 1), 0) % m_sz
    v2 = m_pp < tlen_ref[0]
    col_ok = jax.lax.broadcasted_iota(jnp.int32, (rows, l), 1) < lmax
    ds = jnp.where(v2 & col_ok, ds_ref[0, :, :l], 0.0)

    a_tab = (sl_ref[...] * _SU - su_ref[...] * _SL) * (1.0 / (_SU - _SL)) + \
            (tl_ref[...] * _TU - tu_ref[...] * _TL) * (1.0 / (_TU - _TL))
    b_tab = (su_ref[...] - sl_ref[...]) * (1.0 / (_SU - _SL)) + \
            (tu_ref[...] - tl_ref[...]) * (1.0 / (_TU - _TL))
    a_v = jnp.where(v2, a_tab[1:2, :], a_tab[0:1, :])
    b_v = jnp.where(v2, b_tab[1:2, :], b_tab[0:1, :])

    val = a_v[:, None, :] + b_v[:, None, :] * ds[:, :, None]
    for t in range(bb):
        out_ref[t] = val[t * m_sz:(t + 1) * m_sz]


def _run_shard(traj_loc, mat2, traj_len, lmax_arr,
               emb_su, emb_sl, emb_tu, emb_tl):
    b_sz, m_sz = traj_loc.shape
    n_loc, l_sz = mat2.shape
    e_sz = emb_su.shape[1]
    bb = _BB if b_sz % _BB == 0 else 1
    grid = (b_sz // bb,)
    rows = bb * m_sz

    table = jnp.pad(mat2, ((1, 0), (0, _DPAD - l_sz)))
    n_pairs = b_sz * m_sz
    idx = traj_loc.astype(jnp.int32).reshape(-1)
    ds_rows = _sc_gather(table, idx, n_pairs)
    ds3 = ds_rows.reshape(n_pairs // rows, rows, _DPAD)

    tlen_pp = jnp.repeat(traj_len.astype(jnp.int32), m_sz
                         ).reshape(n_pairs // rows, rows, 1)

    full = lambda s, *refs: (0, 0)

    out = pl.pallas_call(
        _expand_kernel,
        grid_spec=pltpu.PrefetchScalarGridSpec(
            num_scalar_prefetch=1,
            grid=grid,
            in_specs=[
                pl.BlockSpec((1, rows, _DPAD), lambda s, *refs: (s, 0, 0)),
                pl.BlockSpec((1, rows, 1), lambda s, *refs: (s, 0, 0)),
                pl.BlockSpec((2, e_sz), full),
                pl.BlockSpec((2, e_sz), full),
                pl.BlockSpec((2, e_sz), full),
                pl.BlockSpec((2, e_sz), full),
            ],
            out_specs=pl.BlockSpec((bb, m_sz, l_sz, e_sz),
                                   lambda s, *refs: (s, 0, 0, 0)),
        ),
        out_shape=jax.ShapeDtypeStruct((b_sz, m_sz, l_sz, e_sz), jnp.float32),
    )(lmax_arr, ds3, tlen_pp, emb_su, emb_sl, emb_tu, emb_tl)
    return out


def kernel(traj_loc, mat2, vec, traj_len, l_max, emb_su, emb_sl, emb_tu, emb_tl):
    del vec
    b_sz = traj_loc.shape[0]
    lmax_arr = jnp.asarray(l_max, jnp.int32).reshape(1)

    devs = [d for d in jax.devices() if d.platform == "tpu"]
    n_dev = 2 if (len(devs) >= 2 and b_sz % 2 == 0) else 1
    if n_dev == 1:
        return _run_shard(traj_loc, mat2, traj_len, lmax_arr,
                          emb_su, emb_sl, emb_tu, emb_tl)

    mesh = Mesh(np.array(devs[:n_dev]), ("d",))
    fn = shard_map(
        _run_shard, mesh=mesh,
        in_specs=(P("d"), P(), P("d"), P(), P(), P(), P(), P()),
        out_specs=P("d"),
        check_rep=False,
    )
    return fn(traj_loc, mat2, traj_len, lmax_arr,
              emb_su, emb_sl, emb_tu, emb_tl)


# speedup vs baseline: 2.8670x; 2.8670x over previous
import functools

import jax
import jax.numpy as jnp
from jax import lax
from jax.experimental import pallas as pl
from jax.experimental.pallas import tpu as pltpu
from jax.experimental.pallas import tpu_sc as plsc

_SU, _SL, _TU, _TL = 1000.0, 0.0, 500.0, 0.0
_BB = 4
_DPAD = 128


def _pick_rows_per_worker(n_rows):
    for b_per_w in (128, 64, 32, 16, 8):
        if n_rows % b_per_w == 0 and n_rows // b_per_w <= 32:
            return b_per_w
    return None


def _sc_gather(table, idx, n_rows):
    info = plsc.get_sparse_core_info()
    b_per_w = _pick_rows_per_worker(n_rows)
    n_active = n_rows // b_per_w
    d = table.shape[1]
    mesh = plsc.VectorSubcoreMesh(core_axis_name="c", subcore_axis_name="s")

    @functools.partial(
        pl.kernel, mesh=mesh,
        out_type=jax.ShapeDtypeStruct((n_rows, d), jnp.float32),
        scratch_types=[
            pltpu.VMEM((b_per_w,), jnp.int32),
            pltpu.VMEM((b_per_w, d), jnp.float32),
            pltpu.SemaphoreType.DMA,
        ],
    )
    def k(table_hbm, idx_hbm, out_hbm, idx_v, rows_v, sem):
        wid = lax.axis_index("s") * info.num_cores + lax.axis_index("c")

        @pl.when(wid < n_active)
        def _():
            base = wid * b_per_w
            pltpu.sync_copy(idx_hbm.at[pl.ds(base, b_per_w)], idx_v)
            pltpu.async_copy(table_hbm.at[idx_v], rows_v, sem).wait()
            pltpu.sync_copy(rows_v, out_hbm.at[pl.ds(base, b_per_w)])

    return k(table, idx)


def _expand_kernel(lmax_ref,
                   ds_ref, tlen_ref, su_ref, sl_ref, tu_ref, tl_ref,
                   out_ref):
    bb, m_sz, l, e = out_ref.shape
    rows = bb * m_sz

    lmax = lmax_ref[0]
    m_pp = jax.lax.broadcasted_iota(jnp.int32, (rows, 1), 0) % m_sz
    v2 = m_pp < tlen_ref[0]
    col_ok = jax.lax.broadcasted_iota(jnp.int32, (rows, l), 1) < lmax
    ds = jnp.where(v2 & col_ok, ds_ref[0, :, :l], 0.0)

    a_tab = (sl_ref[...] * _SU - su_ref[...] * _SL) * (1.0 / (_SU - _SL)) + \
            (tl_ref[...] * _TU - tu_ref[...] * _TL) * (1.0 / (_TU - _TL))
    b_tab = (su_ref[...] - sl_ref[...]) * (1.0 / (_SU - _SL)) + \
            (tu_ref[...] - tl_ref[...]) * (1.0 / (_TU - _TL))
    a_v = jnp.where(v2, a_tab[1:2, :], a_tab[0:1, :])
    b_v = jnp.where(v2, b_tab[1:2, :], b_tab[0:1, :])

    val = a_v[:, None, :] + b_v[:, None, :] * ds[:, :, None]
    for t in range(bb):
        out_ref[t] = val[t * m_sz:(t + 1) * m_sz]


def kernel(traj_loc, mat2, vec, traj_len, l_max, emb_su, emb_sl, emb_tu, emb_tl):
    del vec
    b_sz, m_sz = traj_loc.shape
    n_loc, l_sz = mat2.shape
    e_sz = emb_su.shape[1]
    bb = _BB if b_sz % _BB == 0 else 1
    grid = (b_sz // bb,)
    rows = bb * m_sz

    table = jnp.pad(mat2, ((1, 0), (0, _DPAD - l_sz)))
    n_pairs = b_sz * m_sz
    idx = traj_loc.astype(jnp.int32).reshape(-1)
    ds_rows = _sc_gather(table, idx, n_pairs)
    ds3 = ds_rows.reshape(n_pairs // rows, rows, _DPAD)

    tlen_pp = jnp.repeat(traj_len.astype(jnp.int32), m_sz
                         ).reshape(n_pairs // rows, rows, 1)

    lmax_arr = jnp.asarray(l_max, jnp.int32).reshape(1)
    full = lambda s, *refs: (0, 0)

    out = pl.pallas_call(
        _expand_kernel,
        grid_spec=pltpu.PrefetchScalarGridSpec(
            num_scalar_prefetch=1,
            grid=grid,
            in_specs=[
                pl.BlockSpec((1, rows, _DPAD), lambda s, *refs: (s, 0, 0)),
                pl.BlockSpec((1, rows, 1), lambda s, *refs: (s, 0, 0)),
                pl.BlockSpec((2, e_sz), full),
                pl.BlockSpec((2, e_sz), full),
                pl.BlockSpec((2, e_sz), full),
                pl.BlockSpec((2, e_sz), full),
            ],
            out_specs=pl.BlockSpec((bb, m_sz, l_sz, e_sz),
                                   lambda s, *refs: (s, 0, 0, 0)),
        ),
        out_shape=jax.ShapeDtypeStruct((b_sz, m_sz, l_sz, e_sz), jnp.float32),
    )(lmax_arr, ds3, tlen_pp, emb_su, emb_sl, emb_tu, emb_tl)
    return out


# speedup vs baseline: 3.1788x; 1.1087x over previous
import functools

import jax
import jax.numpy as jnp
from jax.experimental import pallas as pl
from jax.experimental.pallas import tpu as pltpu

_SU, _SL, _TU, _TL = 1000.0, 0.0, 500.0, 0.0
_BB = 2


def _embed_kernel(loc_ref, len_ref, lmax_ref,
                  mat2_ref, su_ref, sl_ref, tu_ref, tl_ref,
                  out_ref, ds_ref):
    s = pl.program_id(0)
    bb, m_sz, l, e = out_ref.shape
    rows = bb * m_sz

    for t in range(rows):
        idx = loc_ref[s * bb + t // m_sz, t % m_sz] - 1
        ds_ref[pl.ds(t, 1), :] = mat2_ref[pl.ds(idx, 1), :]

    lmax = lmax_ref[0]
    m_pp = jax.lax.broadcasted_iota(jnp.int32, (rows, 1), 0) % m_sz
    tl0 = len_ref[s * bb]
    tl1 = len_ref[s * bb + bb - 1]
    tlen = jnp.where(jax.lax.broadcasted_iota(jnp.int32, (rows, 1), 0) < m_sz,
                     tl0, tl1) if bb == 2 else tl0
    v2 = m_pp < tlen
    col_ok = jax.lax.broadcasted_iota(jnp.int32, (rows, l), 1) < lmax
    ds = jnp.where(v2 & col_ok, ds_ref[...], 0.0)

    a_tab = (sl_ref[...] * _SU - su_ref[...] * _SL) * (1.0 / (_SU - _SL)) + \
            (tl_ref[...] * _TU - tu_ref[...] * _TL) * (1.0 / (_TU - _TL))
    b_tab = (su_ref[...] - sl_ref[...]) * (1.0 / (_SU - _SL)) + \
            (tu_ref[...] - tl_ref[...]) * (1.0 / (_TU - _TL))
    a_v = jnp.where(v2, a_tab[1:2, :], a_tab[0:1, :])
    b_v = jnp.where(v2, b_tab[1:2, :], b_tab[0:1, :])

    val = a_v[:, None, :] + b_v[:, None, :] * ds[:, :, None]
    for t in range(bb):
        out_ref[t] = val[t * m_sz:(t + 1) * m_sz]


def kernel(traj_loc, mat2, vec, traj_len, l_max, emb_su, emb_sl, emb_tu, emb_tl):
    del vec
    b_sz, m_sz = traj_loc.shape
    n_loc, l_sz = mat2.shape
    e_sz = emb_su.shape[1]
    bb = _BB if b_sz % _BB == 0 else 1
    grid = (b_sz // bb,)
    rows = bb * m_sz

    lmax_arr = jnp.asarray(l_max, jnp.int32).reshape(1)
    full = lambda s, *refs: (0, 0)

    out = pl.pallas_call(
        _embed_kernel,
        grid_spec=pltpu.PrefetchScalarGridSpec(
            num_scalar_prefetch=3,
            grid=grid,
            in_specs=[
                pl.BlockSpec((n_loc, l_sz), full),
                pl.BlockSpec((2, e_sz), full),
                pl.BlockSpec((2, e_sz), full),
                pl.BlockSpec((2, e_sz), full),
                pl.BlockSpec((2, e_sz), full),
            ],
            out_specs=pl.BlockSpec((bb, m_sz, l_sz, e_sz),
                                   lambda s, *refs: (s, 0, 0, 0)),
            scratch_shapes=[pltpu.VMEM((rows, l_sz), jnp.float32)],
        ),
        out_shape=jax.ShapeDtypeStruct((b_sz, m_sz, l_sz, e_sz), jnp.float32),
    )(traj_loc.astype(jnp.int32), traj_len.astype(jnp.int32), lmax_arr,
      mat2, emb_su, emb_sl, emb_tu, emb_tl)
    return out
